# baseline (speedup 1.0000x reference)
import functools

import jax
import jax.numpy as jnp
import numpy as np
from jax.experimental import pallas as pl

_B, _H, _W = 8, 256, 256
_K = 16
_EMBED = 256
_NPF = _EMBED // 2


def _or_fold(x, axis):
    n = x.shape[axis]
    while n > 1:
        n //= 2
        if axis == 0:
            x = x[:n] | x[n:]
        else:
            x = x[:, :n] | x[:, n:]
    return x


_IPB = 8


def _geo_kernel(imap_ref, g_ref, pe2_ref, pe3_ref, out_ref):
    g = g_ref[...]
    def b16(v):
        return v.astype(jnp.bfloat16).astype(jnp.float32)

    g0 = b16(g[0:1, :])
    g1 = b16(g[1:2, :])
    two_pi = jnp.float32(2.0 * np.pi)

    ids_col = jax.lax.broadcasted_iota(jnp.int32, (_K, 1), 0) + 1
    xx = jax.lax.broadcasted_iota(jnp.int32, (_K, _W), 1).astype(jnp.float32)

    ii = jax.lax.broadcasted_iota(jnp.int32, (_H, _H), 0)
    jj = jax.lax.broadcasted_iota(jnp.int32, (_H, _H), 1)
    eye = (ii == jj).astype(jnp.float32)

    def norm(v, denom):
        return (v + 0.5) / denom * 2.0 - 1.0

    for i in range(_IPB):
        m = imap_ref[i]
        bits = jnp.left_shift(jnp.int32(1), m)

        colbits = _or_fold(bits, 0)

        bits_t = jax.lax.dot_general(
            bits.astype(jnp.float32), eye,
            (((0,), (0,)), ((), ())),
            preferred_element_type=jnp.float32).astype(jnp.int32)
        rowbits = _or_fold(bits_t, 0)

        colk = jnp.bitwise_and(jnp.right_shift(colbits, ids_col), 1)
        min_x = jnp.min(jnp.where(colk == 1, xx, float(_W)), axis=1,
                        keepdims=True)
        max_x = jnp.max(jnp.where(colk == 1, xx, -1.0), axis=1, keepdims=True)

        rowk = jnp.bitwise_and(jnp.right_shift(rowbits, ids_col), 1)
        min_y = jnp.min(jnp.where(rowk == 1, xx, float(_H)), axis=1,
                        keepdims=True)
        max_y = jnp.max(jnp.where(rowk == 1, xx, -1.0), axis=1, keepdims=True)

        c0x = b16(norm(min_x, float(_W)))
        c0y = b16(norm(min_y, float(_H)))
        c1x = b16(norm(max_x, float(_W)))
        c1y = b16(norm(max_y, float(_H)))

        pe0 = two_pi * (c0x * g0 + c0y * g1)
        pe1 = two_pi * (c1x * g0 + c1y * g1)

        pe2 = pe2_ref[...]
        pe3 = pe3_ref[...]
        r = pl.ds(i * _K, _K)
        out_ref[r, 0:_NPF] = jnp.sin(pe0) + pe2[:, 0:_NPF]
        out_ref[r, _NPF:2 * _NPF] = jnp.cos(pe0) + pe2[:, _NPF:2 * _NPF]
        out_ref[r, 2 * _NPF:3 * _NPF] = jnp.sin(pe1) + pe3[:, 0:_NPF]
        out_ref[r, 3 * _NPF:4 * _NPF] = jnp.cos(pe1) + pe3[:, _NPF:2 * _NPF]


@jax.jit
def _run(instance_map, G, pe2, pe3):
    return pl.pallas_call(
        _geo_kernel,
        grid=(_B // _IPB,),
        in_specs=[
            pl.BlockSpec((_IPB, _H, _W), lambda b: (b, 0, 0)),
            pl.BlockSpec((2, _NPF), lambda b: (0, 0)),
            pl.BlockSpec((1, _EMBED), lambda b: (0, 0)),
            pl.BlockSpec((1, _EMBED), lambda b: (0, 0)),
        ],
        out_specs=pl.BlockSpec((_IPB * _K, 2 * _EMBED), lambda b: (b, 0)),
        out_shape=jax.ShapeDtypeStruct((_B * _K, 2 * _EMBED), jnp.float32),
    )(instance_map, G, pe2, pe3)


def kernel(seg, instance_map, G, pe2, pe3):
    del seg
    return _run(instance_map, G, pe2, pe3)


# speedup vs baseline: 1.0502x; 1.0502x over previous
import functools

import jax
import jax.numpy as jnp
import numpy as np
from jax.experimental import pallas as pl

_B, _H, _W = 8, 256, 256
_K = 16
_EMBED = 256
_NPF = _EMBED // 2


def _or_fold(x, axis):
    n = x.shape[axis]
    while n > 1:
        n //= 2
        if axis == 0:
            x = x[:n] | x[n:]
        else:
            x = x[:, :n] | x[:, n:]
    return x


_IPB = 4


def _geo_kernel(imap_ref, g_ref, pe2_ref, pe3_ref, out_ref):
    g = g_ref[...]
    def b16(v):
        return v.astype(jnp.bfloat16).astype(jnp.float32)

    g0 = b16(g[0:1, :])
    g1 = b16(g[1:2, :])
    two_pi = jnp.float32(2.0 * np.pi)

    ids_col = jax.lax.broadcasted_iota(jnp.int32, (_K, 1), 0) + 1
    xx = jax.lax.broadcasted_iota(jnp.int32, (_K, _W), 1).astype(jnp.float32)

    ii = jax.lax.broadcasted_iota(jnp.int32, (_H, _H), 0)
    jj = jax.lax.broadcasted_iota(jnp.int32, (_H, _H), 1)
    eye = (ii == jj).astype(jnp.float32)

    def norm(v, denom):
        return (v + 0.5) / denom * 2.0 - 1.0

    for i in range(_IPB):
        m = imap_ref[i]
        bits = jnp.left_shift(jnp.int32(1), m)

        colbits = _or_fold(bits, 0)

        bits_t = jax.lax.dot_general(
            bits.astype(jnp.float32), eye,
            (((0,), (0,)), ((), ())),
            preferred_element_type=jnp.float32).astype(jnp.int32)
        rowbits = _or_fold(bits_t, 0)

        colk = jnp.bitwise_and(jnp.right_shift(colbits, ids_col), 1)
        min_x = jnp.min(jnp.where(colk == 1, xx, float(_W)), axis=1,
                        keepdims=True)
        max_x = jnp.max(jnp.where(colk == 1, xx, -1.0), axis=1, keepdims=True)

        rowk = jnp.bitwise_and(jnp.right_shift(rowbits, ids_col), 1)
        min_y = jnp.min(jnp.where(rowk == 1, xx, float(_H)), axis=1,
                        keepdims=True)
        max_y = jnp.max(jnp.where(rowk == 1, xx, -1.0), axis=1, keepdims=True)

        c0x = b16(norm(min_x, float(_W)))
        c0y = b16(norm(min_y, float(_H)))
        c1x = b16(norm(max_x, float(_W)))
        c1y = b16(norm(max_y, float(_H)))

        pe0 = two_pi * (c0x * g0 + c0y * g1)
        pe1 = two_pi * (c1x * g0 + c1y * g1)

        pe2 = pe2_ref[...]
        pe3 = pe3_ref[...]
        r = pl.ds(i * _K, _K)
        out_ref[r, 0:_NPF] = jnp.sin(pe0) + pe2[:, 0:_NPF]
        out_ref[r, _NPF:2 * _NPF] = jnp.cos(pe0) + pe2[:, _NPF:2 * _NPF]
        out_ref[r, 2 * _NPF:3 * _NPF] = jnp.sin(pe1) + pe3[:, 0:_NPF]
        out_ref[r, 3 * _NPF:4 * _NPF] = jnp.cos(pe1) + pe3[:, _NPF:2 * _NPF]


@jax.jit
def _run(instance_map, G, pe2, pe3):
    return pl.pallas_call(
        _geo_kernel,
        grid=(_B // _IPB,),
        in_specs=[
            pl.BlockSpec((_IPB, _H, _W), lambda b: (b, 0, 0)),
            pl.BlockSpec((2, _NPF), lambda b: (0, 0)),
            pl.BlockSpec((1, _EMBED), lambda b: (0, 0)),
            pl.BlockSpec((1, _EMBED), lambda b: (0, 0)),
        ],
        out_specs=pl.BlockSpec((_IPB * _K, 2 * _EMBED), lambda b: (b, 0)),
        out_shape=jax.ShapeDtypeStruct((_B * _K, 2 * _EMBED), jnp.float32),
    )(instance_map, G, pe2, pe3)


def kernel(seg, instance_map, G, pe2, pe3):
    del seg
    return _run(instance_map, G, pe2, pe3)
